# baseline (speedup 1.0000x reference)
import jax
import jax.numpy as jnp
from jax.experimental import pallas as pl
from jax.experimental.pallas import tpu as pltpu


_BS = 256
_NBUF = 8


def _copy_kernel(w_hbm, o_hbm, bufs, in_sems, out_sems, *, bsz, dim, bs,
                 nbuf, nchunks):
    def in_copy(i):
        s = i % nbuf
        return pltpu.make_async_copy(
            w_hbm.at[pl.ds(i * bs, bs), :], bufs.at[s], in_sems.at[s])

    def out_copy(i, b):
        s = i % nbuf
        return pltpu.make_async_copy(
            bufs.at[s],
            o_hbm.at[pl.ds(i * bs, bs), pl.ds(b * dim, dim)],
            out_sems.at[s, b])

    for s in range(min(nbuf, nchunks)):
        in_copy(s).start()
    for i in range(nchunks):
        in_copy(i).wait()
        for b in range(bsz):
            out_copy(i, b).start()
        j = i - (nbuf - 1)
        if j >= 0:
            for b in range(bsz):
                out_copy(j, b).wait()
            if j + nbuf < nchunks:
                in_copy(j + nbuf).start()
    for j in range(max(0, nchunks - nbuf + 1), nchunks):
        for b in range(bsz):
            out_copy(j, b).wait()


def kernel(input, weights):
    seq_len, bsz = input.shape
    init_size, dim = weights.shape
    bs = _BS if seq_len % _BS == 0 else seq_len
    nchunks = seq_len // bs
    nbuf = min(_NBUF, nchunks)
    out = pl.pallas_call(
        lambda w, o, bufs, isem, osem: _copy_kernel(
            w, o, bufs, isem, osem,
            bsz=bsz, dim=dim, bs=bs, nbuf=nbuf, nchunks=nchunks),
        in_specs=[pl.BlockSpec(memory_space=pl.ANY)],
        out_specs=pl.BlockSpec(memory_space=pl.ANY),
        out_shape=jax.ShapeDtypeStruct((seq_len, bsz * dim), weights.dtype),
        scratch_shapes=[
            pltpu.VMEM((nbuf, bs, dim), weights.dtype),
            pltpu.SemaphoreType.DMA((nbuf,)),
            pltpu.SemaphoreType.DMA((nbuf, bsz)),
        ],
    )(weights[:seq_len])
    return out.reshape(seq_len, bsz, dim)


# speedup vs baseline: 2.7714x; 2.7714x over previous
import functools

import jax
import jax.numpy as jnp
from jax import lax
from jax.experimental import pallas as pl
from jax.experimental.pallas import tpu as pltpu
from jax.experimental.pallas import tpu_sc as plsc


_CB = 32
_NBUF = 2


def _sc_body(w_hbm, o_hbm, buf0, buf1, in_sems, out_sems, *, bsz, dim, cb,
             rows_per_w, nchunks, num_subcores):
    bufs = (buf0, buf1)
    c = lax.axis_index("c")
    s = lax.axis_index("s")
    wid = c * num_subcores + s
    base = wid * rows_per_w

    def in_copy(k):
        sl = k % _NBUF
        return pltpu.make_async_copy(
            w_hbm.at[pl.ds(base + k * cb, cb), :], bufs[sl], in_sems.at[sl])

    def out_copy(k, b):
        sl = k % _NBUF
        return pltpu.make_async_copy(
            bufs[sl],
            o_hbm.at[pl.ds(base + k * cb, cb), b, :],
            out_sems.at[sl, b])

    for k in range(min(_NBUF, nchunks)):
        in_copy(k).start()
    for k in range(nchunks):
        in_copy(k).wait()
        for b in range(bsz):
            out_copy(k, b).start()
        if k >= 1:
            for b in range(bsz):
                out_copy(k - 1, b).wait()
            if k + 1 < nchunks:
                in_copy(k + 1).start()
    for b in range(bsz):
        out_copy(nchunks - 1, b).wait()


def kernel(input, weights):
    seq_len, bsz = input.shape
    init_size, dim = weights.shape
    info = plsc.get_sparse_core_info()
    nw = info.num_cores * info.num_subcores
    rows_per_w = seq_len // nw
    cb = _CB if rows_per_w % _CB == 0 else rows_per_w
    nchunks = rows_per_w // cb
    mesh = plsc.VectorSubcoreMesh(core_axis_name="c", subcore_axis_name="s")
    body = functools.partial(
        _sc_body, bsz=bsz, dim=dim, cb=cb, rows_per_w=rows_per_w,
        nchunks=nchunks, num_subcores=info.num_subcores)
    return pl.kernel(
        body,
        out_type=jax.ShapeDtypeStruct((seq_len, bsz, dim), weights.dtype),
        mesh=mesh,
        scratch_types=[
            pltpu.VMEM((cb, dim), weights.dtype),
            pltpu.VMEM((cb, dim), weights.dtype),
            pltpu.SemaphoreType.DMA((_NBUF,)),
            pltpu.SemaphoreType.DMA((_NBUF, bsz)),
        ],
    )(weights[:seq_len])


# speedup vs baseline: 2.7910x; 1.0071x over previous
import functools

import jax
import jax.numpy as jnp
from jax import lax
from jax.experimental import pallas as pl
from jax.experimental.pallas import tpu as pltpu
from jax.experimental.pallas import tpu_sc as plsc


_CB = 32
_NBUF = 3


def _sc_body(w_hbm, o_hbm, bufs, in_sems, out_sems, *, bsz, dim, cb,
             rows_per_w, nchunks, nbuf, num_subcores):
    c = lax.axis_index("c")
    s = lax.axis_index("s")
    wid = c * num_subcores + s
    base = wid * rows_per_w

    def in_copy(k):
        sl = k % nbuf
        return pltpu.make_async_copy(
            w_hbm.at[pl.ds(base + k * cb, cb), :], bufs.at[sl],
            in_sems.at[sl])

    def out_copy(k, b):
        sl = k % nbuf
        return pltpu.make_async_copy(
            bufs.at[sl],
            o_hbm.at[pl.ds(base + k * cb, cb), b, :],
            out_sems.at[sl, b])

    for k in range(min(nbuf, nchunks)):
        in_copy(k).start()
    for k in range(nchunks):
        in_copy(k).wait()
        for b in range(bsz):
            out_copy(k, b).start()
        j = k - (nbuf - 1)
        if j >= 0:
            for b in range(bsz):
                out_copy(j, b).wait()
            if j + nbuf < nchunks:
                in_copy(j + nbuf).start()
    for j in range(max(0, nchunks - nbuf + 1), nchunks):
        for b in range(bsz):
            out_copy(j, b).wait()


def kernel(input, weights):
    seq_len, bsz = input.shape
    init_size, dim = weights.shape
    info = plsc.get_sparse_core_info()
    nw = info.num_cores * info.num_subcores
    rows_per_w = seq_len // nw
    cb = _CB if rows_per_w % _CB == 0 else rows_per_w
    nchunks = rows_per_w // cb
    nbuf = min(_NBUF, nchunks)
    mesh = plsc.VectorSubcoreMesh(core_axis_name="c", subcore_axis_name="s")
    body = functools.partial(
        _sc_body, bsz=bsz, dim=dim, cb=cb, rows_per_w=rows_per_w,
        nchunks=nchunks, nbuf=nbuf, num_subcores=info.num_subcores)
    return pl.kernel(
        body,
        out_type=jax.ShapeDtypeStruct((seq_len, bsz, dim), weights.dtype),
        mesh=mesh,
        scratch_types=[
            pltpu.VMEM((nbuf, cb, dim), weights.dtype),
            pltpu.SemaphoreType.DMA((nbuf,)),
            pltpu.SemaphoreType.DMA((nbuf, bsz)),
        ],
    )(weights[:seq_len])


# speedup vs baseline: 2.7972x; 1.0022x over previous
import functools

import jax
import jax.numpy as jnp
from jax import lax
from jax.experimental import pallas as pl
from jax.experimental.pallas import tpu as pltpu
from jax.experimental.pallas import tpu_sc as plsc


_CB = 32
_NBUF = 3
_ISPLIT = 2


def _sc_body(w_hbm, o_hbm, bufs, in_sems, out_sems, *, bsz, dim, cb,
             rows_per_w, nchunks, nbuf, isplit, num_subcores):
    c = lax.axis_index("c")
    s = lax.axis_index("s")
    wid = c * num_subcores + s
    base = wid * rows_per_w
    colw = dim // isplit

    def in_copy(k, p):
        sl = k % nbuf
        return pltpu.make_async_copy(
            w_hbm.at[pl.ds(base + k * cb, cb), pl.ds(p * colw, colw)],
            bufs.at[sl, :, pl.ds(p * colw, colw)],
            in_sems.at[sl, p])

    def out_copy(k, b):
        sl = k % nbuf
        return pltpu.make_async_copy(
            bufs.at[sl],
            o_hbm.at[pl.ds(base + k * cb, cb), b, :],
            out_sems.at[sl, b])

    for k in range(min(nbuf, nchunks)):
        for p in range(isplit):
            in_copy(k, p).start()
    for k in range(nchunks):
        for p in range(isplit):
            in_copy(k, p).wait()
        for b in range(bsz):
            out_copy(k, b).start()
        j = k - (nbuf - 1)
        if j >= 0:
            for b in range(bsz):
                out_copy(j, b).wait()
            if j + nbuf < nchunks:
                for p in range(isplit):
                    in_copy(j + nbuf, p).start()
    for j in range(max(0, nchunks - nbuf + 1), nchunks):
        for b in range(bsz):
            out_copy(j, b).wait()


def kernel(input, weights):
    seq_len, bsz = input.shape
    init_size, dim = weights.shape
    info = plsc.get_sparse_core_info()
    nw = info.num_cores * info.num_subcores
    rows_per_w = seq_len // nw
    cb = _CB if rows_per_w % _CB == 0 else rows_per_w
    nchunks = rows_per_w // cb
    nbuf = min(_NBUF, nchunks)
    mesh = plsc.VectorSubcoreMesh(core_axis_name="c", subcore_axis_name="s")
    body = functools.partial(
        _sc_body, bsz=bsz, dim=dim, cb=cb, rows_per_w=rows_per_w,
        nchunks=nchunks, nbuf=nbuf, isplit=_ISPLIT,
        num_subcores=info.num_subcores)
    return pl.kernel(
        body,
        out_type=jax.ShapeDtypeStruct((seq_len, bsz, dim), weights.dtype),
        mesh=mesh,
        scratch_types=[
            pltpu.VMEM((nbuf, cb, dim), weights.dtype),
            pltpu.SemaphoreType.DMA((nbuf, _ISPLIT)),
            pltpu.SemaphoreType.DMA((nbuf, bsz)),
        ],
    )(weights[:seq_len])
